# baseline (speedup 1.0000x reference)
import functools

import jax
import jax.numpy as jnp
from jax import lax
from jax.experimental import pallas as pl
from jax.experimental.pallas import tpu as pltpu
from jax.experimental.pallas import tpu_sc as plsc

NV = 10000
RP = 10240
SH = RP // 2
HD = 256
BN = 1024
CH = 128
NT = 16
TR = SH // NT


def _sc_segsum(data, gidx2, sidx2, zeros):
    Pp = gidx2.shape[1]
    per_tile = Pp // NT
    nch = per_tile // CH
    mesh = plsc.VectorSubcoreMesh(core_axis_name="c", subcore_axis_name="s",
                                  num_cores=2, num_subcores=NT)

    @functools.partial(
        pl.kernel,
        mesh=mesh,
        out_type=jax.ShapeDtypeStruct((RP, HD), jnp.float32),
        scratch_types=[
            pltpu.VMEM((2, CH, HD), jnp.float32),
            pltpu.VMEM((2, CH), jnp.int32),
            pltpu.VMEM((2, CH), jnp.int32),
            pltpu.SemaphoreType.DMA,
            pltpu.SemaphoreType.DMA,
        ],
    )
    def k(data_h, g_h, s_h, z_h, out_h, db, gb, sb, sem0, sem1):
        c = lax.axis_index("c")
        t = lax.axis_index("s")
        sems = (sem0, sem1)
        pltpu.sync_copy(z_h, out_h.at[pl.ds(c * SH + t * TR, TR)])
        plsc.subcore_barrier()
        base = t * per_tile
        descs = {}

        def start(j, b):
            off = base + j * CH
            pltpu.sync_copy(g_h.at[c, pl.ds(off, CH)], gb.at[b])
            pltpu.sync_copy(s_h.at[c, pl.ds(off, CH)], sb.at[b])
            descs[b] = pltpu.async_copy(data_h.at[gb.at[b]], db.at[b], sems[b])

        start(0, 0)
        for j in range(nch):
            if j + 1 < nch:
                start(j + 1, (j + 1) % 2)
            b = j % 2
            descs[b].wait()
            pltpu.sync_copy(db.at[b], out_h.at[sb.at[b]], add=True)

    return k(data, gidx2, sidx2, zeros)


def _pairs(g, d, Pp, zr):
    P = g.shape[0]
    pad = Pp - P
    gp = jnp.concatenate([g.astype(jnp.int32),
                          jnp.full((pad,), zr, jnp.int32)])
    dp = jnp.concatenate([d.astype(jnp.int32),
                          jnp.arange(pad, dtype=jnp.int32) % SH])
    own = dp // SH
    g2 = jnp.stack([jnp.where(own == 0, gp, zr), jnp.where(own == 1, gp, zr)])
    sloc = dp % SH
    s2 = jnp.stack([sloc, SH + sloc])
    return g2, s2


def _tc_mlp(h, s, eo, Wa, ba, Wb, bb, second=None,
            relu_h=False, relu2_in=False):
    has_s = s is not None
    has_eo = eo is not None
    has2 = second is not None
    od = Wb.shape[1]

    def body(*refs):
        it = iter(refs)
        h_ref = next(it)
        s_ref = next(it) if has_s else None
        eo_ref = next(it) if has_eo else None
        Wa_ref, ba_ref, Wb_ref, bb_ref = next(it), next(it), next(it), next(it)
        if has2:
            W2a_ref, b2a_ref, W2b_ref, b2b_ref = (next(it), next(it),
                                                  next(it), next(it))
        o1_ref = next(it)
        o2_ref = next(it) if has2 else None

        i = pl.program_id(0)
        hv = h_ref[...]
        if relu_h:
            hv = jnp.maximum(hv, 0.0)
        if has_s:
            sv = s_ref[...]
            if has_eo:
                sv = sv * (eo_ref[...] > 2).astype(jnp.float32)
            hv = jnp.concatenate([hv, sv], axis=1)
        z = jnp.maximum(
            jnp.dot(hv, Wa_ref[...], preferred_element_type=jnp.float32)
            + ba_ref[...], 0.0)
        y1 = jnp.dot(z, Wb_ref[...],
                     preferred_element_type=jnp.float32) + bb_ref[...]
        rows = i * BN + lax.broadcasted_iota(jnp.int32, (BN, 1), 0)
        valid = rows < NV
        o1_ref[...] = jnp.where(valid, y1, 0.0)
        if has2:
            t = jnp.maximum(y1, 0.0) if relu2_in else y1
            z2 = jnp.maximum(
                jnp.dot(t, W2a_ref[...], preferred_element_type=jnp.float32)
                + b2a_ref[...], 0.0)
            y2 = jnp.dot(z2, W2b_ref[...],
                         preferred_element_type=jnp.float32) + b2b_ref[...]
            o2_ref[...] = jnp.where(valid, y2, 0.0)

    def row_spec(a):
        nd = a.ndim
        return pl.BlockSpec((BN,) + a.shape[1:],
                            lambda i, _nd=nd: (i,) + (0,) * (_nd - 1))

    def full_spec(a):
        nd = a.ndim
        return pl.BlockSpec(a.shape, lambda i, _nd=nd: (0,) * _nd)

    args = [h]
    specs = [row_spec(h)]
    if has_s:
        args.append(s)
        specs.append(row_spec(s))
    if has_eo:
        args.append(eo)
        specs.append(row_spec(eo))
    for w in (Wa, ba, Wb, bb):
        args.append(w)
        specs.append(full_spec(w))
    if has2:
        for w in second:
            args.append(w)
            specs.append(full_spec(w))

    out_shape = [jax.ShapeDtypeStruct((RP, od), jnp.float32)]
    out_specs = [pl.BlockSpec((BN, od), lambda i: (i, 0))]
    if has2:
        out_shape.append(jax.ShapeDtypeStruct((RP, HD), jnp.float32))
        out_specs.append(pl.BlockSpec((BN, HD), lambda i: (i, 0)))

    res = pl.pallas_call(
        body,
        grid=(RP // BN,),
        in_specs=specs,
        out_specs=out_specs,
        out_shape=out_shape,
        compiler_params=pltpu.CompilerParams(
            dimension_semantics=("arbitrary",)),
    )(*args)
    return res if has2 else res[0]


def kernel(x, edge_attr, edge_index0, edge_index1, n_e, e_order, batch,
           atom_emb, bond_emb, W1a, b1a, W1b, b1b, W2a, b2a, W2b, b2b,
           W3a, b3a, W3b, b3b, W4a, b4a, W4b, b4b, Wo1, bo1, Wo2, bo2):
    f32 = jnp.float32
    i32 = jnp.int32
    V, E = edge_index0, edge_index1
    zeros = jnp.zeros((TR, HD), f32)

    at_ext = jnp.concatenate([atom_emb.reshape(9 * 100, HD).astype(f32),
                              jnp.zeros((4, HD), f32)])
    ZT = 900
    x_p = jnp.pad(x, ((0, RP - NV), (0, 0)))
    g_all = x_p + (jnp.arange(9, dtype=i32) * 100)[None, :]
    g_all = jnp.where((jnp.arange(RP) < NV)[:, None], g_all, ZT)
    PpA = 47104
    padA = PpA - 9 * SH
    gA2 = jnp.stack([
        jnp.concatenate([g_all[:SH].reshape(-1),
                         jnp.full((padA,), ZT, i32)]),
        jnp.concatenate([g_all[SH:].reshape(-1),
                         jnp.full((padA,), ZT, i32)]),
    ])
    sA = jnp.concatenate([jnp.repeat(jnp.arange(SH, dtype=i32), 9),
                          jnp.arange(padA, dtype=i32) % SH])
    sA2 = jnp.stack([sA, SH + sA])

    bd_ext = jnp.concatenate([bond_emb.astype(f32), jnp.zeros((10, HD), f32)])
    ZR = NV
    gB2, sB2 = _pairs(edge_attr[:, 0], jnp.arange(NV, dtype=i32), RP, 6)
    gE2, sE2 = _pairs(V, E, 32768, ZR)
    gV2, sV2 = _pairs(E, V, 32768, ZR)
    gP2, sP2 = _pairs(jnp.arange(NV, dtype=i32), batch, RP, ZR)

    eo_p = jnp.pad(e_order, (0, RP - NV)).reshape(RP, 1)
    b1a_, b1b_ = b1a.reshape(1, HD), b1b.reshape(1, HD)
    b2a_, b2b_ = b2a.reshape(1, HD), b2b.reshape(1, HD)
    b3a_, b3b_ = b3a.reshape(1, HD), b3b.reshape(1, HD)
    b4a_, b4b_ = b4a.reshape(1, HD), b4b.reshape(1, HD)
    bo1_, bo2_ = bo1.reshape(1, HD), bo2.reshape(1, 1)

    xh = _sc_segsum(at_ext, gA2, sA2, zeros)
    eh = _sc_segsum(bd_ext, gB2, sB2, zeros)
    A = _tc_mlp(xh, None, None, W1a, b1a_, W1b, b1b_)
    for i in range(3):
        Xe = _sc_segsum(A, gE2, sE2, zeros)
        eh_new, Bm = _tc_mlp(eh, Xe, None, W2a, b2a_, W2b, b2b_,
                             second=(W3a, b3a_, W3b, b3b_),
                             relu_h=(i > 0), relu2_in=False)
        Xv = _sc_segsum(Bm, gV2, sV2, zeros)
        if i < 2:
            xh, A = _tc_mlp(xh, Xv, None, W4a, b4a_, W4b, b4b_,
                            second=(W1a, b1a_, W1b, b1b_),
                            relu_h=(i > 0), relu2_in=True)
        else:
            xh = _tc_mlp(xh, Xv, None, W4a, b4a_, W4b, b4b_, relu_h=True)
        eh = eh_new

    xg = _sc_segsum(xh, gP2, sP2, zeros)
    out = _tc_mlp(xg, eh, eo_p, Wo1, bo1_, Wo2, bo2_)
    return out[:NV, 0]


# speedup vs baseline: 3.0155x; 3.0155x over previous
import functools

import jax
import jax.numpy as jnp
from jax import lax
from jax.experimental import pallas as pl
from jax.experimental.pallas import tpu as pltpu
from jax.experimental.pallas import tpu_sc as plsc

NV = 10000
RP = 10240
HD = 256
BN = 1024
BS = 128
L2 = 1024
NW = 32
SENT = 1 << 30


def _sc_gather(data, gidx):
    Pp = gidx.shape[0]
    per_tile = Pp // NW
    ch = per_tile if per_tile <= 128 else 128
    while per_tile % ch:
        ch //= 2
    nch = per_tile // ch
    mesh = plsc.VectorSubcoreMesh(core_axis_name="c", subcore_axis_name="s",
                                  num_cores=2, num_subcores=16)

    @functools.partial(
        pl.kernel,
        mesh=mesh,
        out_type=jax.ShapeDtypeStruct((Pp, HD), jnp.float32),
        scratch_types=[
            pltpu.VMEM((ch, HD), jnp.float32),
            pltpu.VMEM((ch, HD), jnp.float32),
            pltpu.VMEM((ch,), jnp.int32),
            pltpu.VMEM((ch,), jnp.int32),
            pltpu.SemaphoreType.DMA,
            pltpu.SemaphoreType.DMA,
        ],
    )
    def k(data_h, g_h, out_h, db0, db1, gb0, gb1, sem0, sem1):
        c = lax.axis_index("c")
        t = lax.axis_index("s")
        w = c * 16 + t
        dbs, gbs, sems = (db0, db1), (gb0, gb1), (sem0, sem1)
        base = w * per_tile
        descs = {}

        def start(j, b):
            pltpu.sync_copy(g_h.at[pl.ds(base + j * ch, ch)], gbs[b])
            descs[b] = pltpu.async_copy(data_h.at[gbs[b]], dbs[b], sems[b])

        start(0, 0)
        for j in range(nch):
            if j + 1 < nch:
                start(j + 1, (j + 1) % 2)
            b = j % 2
            descs[b].wait()
            pltpu.sync_copy(dbs[b], out_h.at[pl.ds(base + j * ch, ch)])

    return k(data, gidx)


def _tc_onehot(x, ranks, bs):
    Pp = x.shape[0]

    def body(x_ref, r_ref, o_ref):
        xv = x_ref[...]
        rv = r_ref[...]
        kcol = lax.broadcasted_iota(jnp.int32, (bs, bs), 1)
        oh = (rv == kcol).astype(jnp.float32)
        o_ref[...] = lax.dot_general(
            oh, xv, (((0,), (0,)), ((), ())),
            preferred_element_type=jnp.float32)

    return pl.pallas_call(
        body,
        grid=(Pp // bs,),
        in_specs=[pl.BlockSpec((bs, HD), lambda i: (i, 0)),
                  pl.BlockSpec((bs, 1), lambda i: (i, 0))],
        out_specs=pl.BlockSpec((bs, HD), lambda i: (i, 0)),
        out_shape=jax.ShapeDtypeStruct((Pp, HD), jnp.float32),
        compiler_params=pltpu.CompilerParams(
            dimension_semantics=("arbitrary",)),
    )(x, ranks)


def _tc_sum9(x9):
    def body(x_ref, o_ref):
        o_ref[...] = jnp.sum(x_ref[...].reshape(BN, 9, HD), axis=1)

    return pl.pallas_call(
        body,
        grid=(RP // BN,),
        in_specs=[pl.BlockSpec((9 * BN, HD), lambda i: (i, 0))],
        out_specs=pl.BlockSpec((BN, HD), lambda i: (i, 0)),
        out_shape=jax.ShapeDtypeStruct((RP, HD), jnp.float32),
        compiler_params=pltpu.CompilerParams(
            dimension_semantics=("arbitrary",)),
    )(x9)


def _tc_mlp(h, s, s2, eo, Wa, ba, Wb, bb, second=None,
            relu_h=False, relu2_in=False, h2=None):
    has_s = s is not None
    has_s2 = s2 is not None
    has_h2 = h2 is not None
    has_eo = eo is not None
    has2 = second is not None
    od = Wb.shape[1]

    def body(*refs):
        it = iter(refs)
        h_ref = next(it)
        h2_ref = next(it) if has_h2 else None
        s_ref = next(it) if has_s else None
        s2_ref = next(it) if has_s2 else None
        eo_ref = next(it) if has_eo else None
        Wa_ref, ba_ref, Wb_ref, bb_ref = next(it), next(it), next(it), next(it)
        if has2:
            W2a_ref, b2a_ref, W2b_ref, b2b_ref = (next(it), next(it),
                                                  next(it), next(it))
        o1_ref = next(it)
        o2_ref = next(it) if has2 else None

        i = pl.program_id(0)
        hv = h_ref[...]
        if has_h2:
            hv = hv + h2_ref[...]
        if relu_h:
            hv = jnp.maximum(hv, 0.0)
        if has_s:
            sv = s_ref[...]
            if has_s2:
                sv = sv + s2_ref[...]
            if has_eo:
                sv = sv * (eo_ref[...] > 2).astype(jnp.float32)
            hv = jnp.concatenate([hv, sv], axis=1)
        z = jnp.maximum(
            jnp.dot(hv, Wa_ref[...], preferred_element_type=jnp.float32)
            + ba_ref[...], 0.0)
        y1 = jnp.dot(z, Wb_ref[...],
                     preferred_element_type=jnp.float32) + bb_ref[...]
        rows = i * BN + lax.broadcasted_iota(jnp.int32, (BN, 1), 0)
        valid = rows < NV
        o1_ref[...] = jnp.where(valid, y1, 0.0)
        if has2:
            t = jnp.maximum(y1, 0.0) if relu2_in else y1
            z2 = jnp.maximum(
                jnp.dot(t, W2a_ref[...], preferred_element_type=jnp.float32)
                + b2a_ref[...], 0.0)
            y2 = jnp.dot(z2, W2b_ref[...],
                         preferred_element_type=jnp.float32) + b2b_ref[...]
            o2_ref[...] = jnp.where(valid, y2, 0.0)

    def row_spec(a):
        nd = a.ndim
        return pl.BlockSpec((BN,) + a.shape[1:],
                            lambda i, _nd=nd: (i,) + (0,) * (_nd - 1))

    def full_spec(a):
        nd = a.ndim
        return pl.BlockSpec(a.shape, lambda i, _nd=nd: (0,) * _nd)

    args = [h]
    specs = [row_spec(h)]
    for extra, flag in ((h2, has_h2), (s, has_s), (s2, has_s2), (eo, has_eo)):
        if flag:
            args.append(extra)
            specs.append(row_spec(extra))
    for w in (Wa, ba, Wb, bb):
        args.append(w)
        specs.append(full_spec(w))
    if has2:
        for w in second:
            args.append(w)
            specs.append(full_spec(w))

    out_shape = [jax.ShapeDtypeStruct((RP, od), jnp.float32)]
    out_specs = [pl.BlockSpec((BN, od), lambda i: (i, 0))]
    if has2:
        out_shape.append(jax.ShapeDtypeStruct((RP, HD), jnp.float32))
        out_specs.append(pl.BlockSpec((BN, HD), lambda i: (i, 0)))

    res = pl.pallas_call(
        body,
        grid=(RP // BN,),
        in_specs=specs,
        out_specs=out_specs,
        out_shape=out_shape,
        compiler_params=pltpu.CompilerParams(
            dimension_semantics=("arbitrary",)),
    )(*args)
    return res if has2 else res[0]


def _plan(dsort, Pp):
    i32 = jnp.int32
    nb = Pp // BS
    pidx = jnp.arange(Pp, dtype=i32)
    first = jnp.concatenate([jnp.ones((1,), bool), dsort[1:] != dsort[:-1]])
    newf = first | (pidx % BS == 0)
    cs = jnp.cumsum(newf.astype(i32))
    rank = cs - cs[(pidx // BS) * BS]

    bstart = jnp.arange(nb, dtype=i32) * BS
    bend = bstart + BS - 1
    dfirst = dsort[bstart]
    dlast = dsort[bend]
    rlast = rank[bend]
    single = dfirst == dlast

    zslot = (nb - 1) * BS + 1 + (jnp.arange(L2, dtype=i32) % (BS - 1))

    l2d = jnp.stack([dfirst, dlast], 1).reshape(-1)
    l2g = jnp.stack([jnp.where(single, zslot[:nb], bstart),
                     bstart + rlast], 1).reshape(-1)
    pad2 = L2 - 2 * nb
    l2d = jnp.concatenate([l2d, SENT + 1 + jnp.arange(pad2, dtype=i32)])
    l2gptr = jnp.concatenate([l2g, zslot[:pad2]])
    f2 = jnp.concatenate([jnp.ones((1,), bool), l2d[1:] != l2d[:-1]])
    cs2 = jnp.cumsum(f2.astype(i32))
    ranks2 = cs2 - 1

    r = jnp.arange(RP, dtype=i32)
    start = jnp.searchsorted(dsort, r).astype(i32)
    end = jnp.searchsorted(dsort, r, side="right").astype(i32)
    nonempty = end > start
    startc = jnp.minimum(start, Pp - 1)
    isb = jnp.zeros((RP,), bool).at[jnp.where(dfirst < RP, dfirst, RP)].set(
        True, mode="drop")
    isb = isb.at[jnp.where(dlast < RP, dlast, RP)].set(True, mode="drop")
    isb = isb & nonempty
    in1 = nonempty & ~isb
    slot1 = (startc // BS) * BS + rank[startc]
    ptr1 = jnp.where(in1, slot1, zslot[r % L2])
    r2of = jnp.zeros((RP,), i32).at[jnp.where(l2d < RP, l2d, RP)].set(
        ranks2, mode="drop")
    ptr2 = jnp.where(isb, r2of, ranks2[2 * nb + (r % pad2)])
    return (rank.reshape(Pp, 1), l2gptr, ranks2.reshape(L2, 1),
            ptr1, ptr2)


def _segsum(data, gptr, plan):
    ranks, l2gptr, ranks2, ptr1, ptr2 = plan
    xs = _sc_gather(data, gptr)
    lvl1 = _tc_onehot(xs, ranks, BS)
    l2in = _sc_gather(lvl1, l2gptr)
    lvl2 = _tc_onehot(l2in, ranks2, L2)
    s1 = _sc_gather(lvl1, ptr1)
    s2 = _sc_gather(lvl2, ptr2)
    return s1, s2


def _sort_pairs(g, d, Pp, zr0, nzr):
    i32 = jnp.int32
    order = jnp.argsort(d)
    gs = g.astype(i32)[order]
    ds = d.astype(i32)[order]
    pad = Pp - g.shape[0]
    zs = zr0 + (jnp.arange(pad, dtype=i32) % nzr)
    gptr = jnp.concatenate([gs, zs])
    dsort = jnp.concatenate([ds, jnp.full((pad,), SENT, i32)])
    return gptr, dsort


def kernel(x, edge_attr, edge_index0, edge_index1, n_e, e_order, batch,
           atom_emb, bond_emb, W1a, b1a, W1b, b1b, W2a, b2a, W2b, b2b,
           W3a, b3a, W3b, b3b, W4a, b4a, W4b, b4b, Wo1, bo1, Wo2, bo2):
    f32, i32 = jnp.float32, jnp.int32
    V, E = edge_index0.astype(i32), edge_index1.astype(i32)
    ZR, NZ = NV, RP - NV

    at_ext = jnp.concatenate([atom_emb.reshape(900, HD).astype(f32),
                              jnp.zeros((124, HD), f32)])
    x_p = jnp.pad(x.astype(i32), ((0, RP - NV), (0, 0)))
    g9 = x_p + (jnp.arange(9, dtype=i32) * 100)[None, :]
    zs_n = 900 + (jnp.arange(RP, dtype=i32) % 124)[:, None]
    g9 = jnp.where((jnp.arange(RP) < NV)[:, None], g9, zs_n).reshape(-1)

    bd_ext = jnp.concatenate([bond_emb.astype(f32), jnp.zeros((58, HD), f32)])
    gbd = jnp.concatenate([edge_attr[:, 0].astype(i32),
                           6 + (jnp.arange(RP - NV, dtype=i32) % 58)])

    gE, dE = _sort_pairs(V, E, 32768, ZR, NZ)
    planE = _plan(dE, 32768)
    gV, dV = _sort_pairs(E, V, 32768, ZR, NZ)
    planV = _plan(dV, 32768)
    dB = jnp.concatenate([batch.astype(i32),
                          jnp.full((RP - NV,), SENT, i32)])
    planB = _plan(dB, RP)

    eo_p = jnp.pad(e_order.astype(i32), (0, RP - NV)).reshape(RP, 1)
    b1a_, b1b_ = b1a.reshape(1, HD), b1b.reshape(1, HD)
    b2a_, b2b_ = b2a.reshape(1, HD), b2b.reshape(1, HD)
    b3a_, b3b_ = b3a.reshape(1, HD), b3b.reshape(1, HD)
    b4a_, b4b_ = b4a.reshape(1, HD), b4b.reshape(1, HD)
    bo1_, bo2_ = bo1.reshape(1, HD), bo2.reshape(1, 1)

    xh = _tc_sum9(_sc_gather(at_ext, g9))
    eh = _sc_gather(bd_ext, gbd)
    A = _tc_mlp(xh, None, None, None, W1a, b1a_, W1b, b1b_)
    for i in range(3):
        e1, e2 = _segsum(A, gE, planE)
        eh_new, Bm = _tc_mlp(eh, e1, e2, None, W2a, b2a_, W2b, b2b_,
                             second=(W3a, b3a_, W3b, b3b_),
                             relu_h=(i > 0), relu2_in=False)
        v1, v2 = _segsum(Bm, gV, planV)
        if i < 2:
            xh, A = _tc_mlp(xh, v1, v2, None, W4a, b4a_, W4b, b4b_,
                            second=(W1a, b1a_, W1b, b1b_),
                            relu_h=(i > 0), relu2_in=True)
        else:
            xh = _tc_mlp(xh, v1, v2, None, W4a, b4a_, W4b, b4b_, relu_h=True)
        eh = eh_new

    ranksB, l2gB, ranks2B, ptr1B, ptr2B = planB
    lvl1 = _tc_onehot(xh, ranksB, BS)
    lvl2 = _tc_onehot(_sc_gather(lvl1, l2gB), ranks2B, L2)
    g1 = _sc_gather(lvl1, ptr1B)
    g2 = _sc_gather(lvl2, ptr2B)
    out = _tc_mlp(g1, eh, None, eo_p, Wo1, bo1_, Wo2, bo2_, h2=g2)
    return out[:NV, 0]


# speedup vs baseline: 3.0239x; 1.0028x over previous
import functools

import jax
import jax.numpy as jnp
from jax import lax
from jax.experimental import pallas as pl
from jax.experimental.pallas import tpu as pltpu
from jax.experimental.pallas import tpu_sc as plsc

NV = 10000
RP = 10240
HD = 256
BN = 1024
BS = 128
L2 = 1024
NW = 32
SENT = 1 << 30


def _sc_gather(data, gidx):
    Pp = gidx.shape[0]
    per_tile = Pp // NW
    ch = per_tile if per_tile <= 128 else 128
    while per_tile % ch:
        ch //= 2
    nch = per_tile // ch
    mesh = plsc.VectorSubcoreMesh(core_axis_name="c", subcore_axis_name="s",
                                  num_cores=2, num_subcores=16)

    nbuf = 3 if ch == 128 else (4 if nch >= 4 else 2)
    mems = []
    for _ in range(nbuf):
        mems.append(pltpu.VMEM((ch, HD), jnp.float32))
    for _ in range(nbuf):
        mems.append(pltpu.VMEM((ch,), jnp.int32))
    for _ in range(nbuf):
        mems.append(pltpu.SemaphoreType.DMA)

    @functools.partial(
        pl.kernel,
        mesh=mesh,
        out_type=jax.ShapeDtypeStruct((Pp, HD), jnp.float32),
        scratch_types=mems,
    )
    def k(data_h, g_h, out_h, *bufs):
        c = lax.axis_index("c")
        t = lax.axis_index("s")
        w = c * 16 + t
        dbs = bufs[:nbuf]
        gbs = bufs[nbuf:2 * nbuf]
        sems = bufs[2 * nbuf:]
        base = w * per_tile
        descs = {}

        def start(j):
            b = j % nbuf
            pltpu.sync_copy(g_h.at[pl.ds(base + j * ch, ch)], gbs[b])
            descs[b] = pltpu.async_copy(data_h.at[gbs[b]], dbs[b], sems[b])

        for j in range(min(nbuf - 1, nch)):
            start(j)
        for j in range(nch):
            if j + nbuf - 1 < nch:
                start(j + nbuf - 1)
            b = j % nbuf
            descs[b].wait()
            pltpu.sync_copy(dbs[b], out_h.at[pl.ds(base + j * ch, ch)])

    return k(data, gidx)


def _tc_onehot(x, ranks, bs):
    Pp = x.shape[0]

    def body(x_ref, r_ref, o_ref):
        xv = x_ref[...]
        rv = r_ref[...]
        kcol = lax.broadcasted_iota(jnp.int32, (bs, bs), 1)
        oh = (rv == kcol).astype(jnp.float32)
        o_ref[...] = lax.dot_general(
            oh, xv, (((0,), (0,)), ((), ())),
            preferred_element_type=jnp.float32)

    return pl.pallas_call(
        body,
        grid=(Pp // bs,),
        in_specs=[pl.BlockSpec((bs, HD), lambda i: (i, 0)),
                  pl.BlockSpec((bs, 1), lambda i: (i, 0))],
        out_specs=pl.BlockSpec((bs, HD), lambda i: (i, 0)),
        out_shape=jax.ShapeDtypeStruct((Pp, HD), jnp.float32),
        compiler_params=pltpu.CompilerParams(
            dimension_semantics=("arbitrary",)),
    )(x, ranks)


def _tc_sum9(x9):
    def body(x_ref, o_ref):
        o_ref[...] = jnp.sum(x_ref[...].reshape(BN, 9, HD), axis=1)

    return pl.pallas_call(
        body,
        grid=(RP // BN,),
        in_specs=[pl.BlockSpec((9 * BN, HD), lambda i: (i, 0))],
        out_specs=pl.BlockSpec((BN, HD), lambda i: (i, 0)),
        out_shape=jax.ShapeDtypeStruct((RP, HD), jnp.float32),
        compiler_params=pltpu.CompilerParams(
            dimension_semantics=("arbitrary",)),
    )(x9)


def _tc_mlp(h, s, s2, eo, Wa, ba, Wb, bb, second=None,
            relu_h=False, relu2_in=False, h2=None):
    has_s = s is not None
    has_s2 = s2 is not None
    has_h2 = h2 is not None
    has_eo = eo is not None
    has2 = second is not None
    od = Wb.shape[1]

    def body(*refs):
        it = iter(refs)
        h_ref = next(it)
        h2_ref = next(it) if has_h2 else None
        s_ref = next(it) if has_s else None
        s2_ref = next(it) if has_s2 else None
        eo_ref = next(it) if has_eo else None
        Wa_ref, ba_ref, Wb_ref, bb_ref = next(it), next(it), next(it), next(it)
        if has2:
            W2a_ref, b2a_ref, W2b_ref, b2b_ref = (next(it), next(it),
                                                  next(it), next(it))
        o1_ref = next(it)
        o2_ref = next(it) if has2 else None

        i = pl.program_id(0)
        hv = h_ref[...]
        if has_h2:
            hv = hv + h2_ref[...]
        if relu_h:
            hv = jnp.maximum(hv, 0.0)
        if has_s:
            sv = s_ref[...]
            if has_s2:
                sv = sv + s2_ref[...]
            if has_eo:
                sv = sv * (eo_ref[...] > 2).astype(jnp.float32)
            hv = jnp.concatenate([hv, sv], axis=1)
        z = jnp.maximum(
            jnp.dot(hv, Wa_ref[...], preferred_element_type=jnp.float32)
            + ba_ref[...], 0.0)
        y1 = jnp.dot(z, Wb_ref[...],
                     preferred_element_type=jnp.float32) + bb_ref[...]
        rows = i * BN + lax.broadcasted_iota(jnp.int32, (BN, 1), 0)
        valid = rows < NV
        o1_ref[...] = jnp.where(valid, y1, 0.0)
        if has2:
            t = jnp.maximum(y1, 0.0) if relu2_in else y1
            z2 = jnp.maximum(
                jnp.dot(t, W2a_ref[...], preferred_element_type=jnp.float32)
                + b2a_ref[...], 0.0)
            y2 = jnp.dot(z2, W2b_ref[...],
                         preferred_element_type=jnp.float32) + b2b_ref[...]
            o2_ref[...] = jnp.where(valid, y2, 0.0)

    def row_spec(a):
        nd = a.ndim
        return pl.BlockSpec((BN,) + a.shape[1:],
                            lambda i, _nd=nd: (i,) + (0,) * (_nd - 1))

    def full_spec(a):
        nd = a.ndim
        return pl.BlockSpec(a.shape, lambda i, _nd=nd: (0,) * _nd)

    args = [h]
    specs = [row_spec(h)]
    for extra, flag in ((h2, has_h2), (s, has_s), (s2, has_s2), (eo, has_eo)):
        if flag:
            args.append(extra)
            specs.append(row_spec(extra))
    for w in (Wa, ba, Wb, bb):
        args.append(w)
        specs.append(full_spec(w))
    if has2:
        for w in second:
            args.append(w)
            specs.append(full_spec(w))

    out_shape = [jax.ShapeDtypeStruct((RP, od), jnp.float32)]
    out_specs = [pl.BlockSpec((BN, od), lambda i: (i, 0))]
    if has2:
        out_shape.append(jax.ShapeDtypeStruct((RP, HD), jnp.float32))
        out_specs.append(pl.BlockSpec((BN, HD), lambda i: (i, 0)))

    res = pl.pallas_call(
        body,
        grid=(RP // BN,),
        in_specs=specs,
        out_specs=out_specs,
        out_shape=out_shape,
        compiler_params=pltpu.CompilerParams(
            dimension_semantics=("arbitrary",)),
    )(*args)
    return res if has2 else res[0]


def _plan(dsort, Pp):
    i32 = jnp.int32
    nb = Pp // BS
    pidx = jnp.arange(Pp, dtype=i32)
    first = jnp.concatenate([jnp.ones((1,), bool), dsort[1:] != dsort[:-1]])
    newf = first | (pidx % BS == 0)
    cs = jnp.cumsum(newf.astype(i32))
    rank = cs - cs[(pidx // BS) * BS]

    bstart = jnp.arange(nb, dtype=i32) * BS
    bend = bstart + BS - 1
    dfirst = dsort[bstart]
    dlast = dsort[bend]
    rlast = rank[bend]
    single = dfirst == dlast

    zslot = (nb - 1) * BS + 1 + (jnp.arange(L2, dtype=i32) % (BS - 1))

    l2d = jnp.stack([dfirst, dlast], 1).reshape(-1)
    l2g = jnp.stack([jnp.where(single, zslot[:nb], bstart),
                     bstart + rlast], 1).reshape(-1)
    pad2 = L2 - 2 * nb
    l2d = jnp.concatenate([l2d, SENT + 1 + jnp.arange(pad2, dtype=i32)])
    l2gptr = jnp.concatenate([l2g, zslot[:pad2]])
    f2 = jnp.concatenate([jnp.ones((1,), bool), l2d[1:] != l2d[:-1]])
    cs2 = jnp.cumsum(f2.astype(i32))
    ranks2 = cs2 - 1

    r = jnp.arange(RP, dtype=i32)
    start = jnp.searchsorted(dsort, r).astype(i32)
    end = jnp.searchsorted(dsort, r, side="right").astype(i32)
    nonempty = end > start
    startc = jnp.minimum(start, Pp - 1)
    isb = jnp.zeros((RP,), bool).at[jnp.where(dfirst < RP, dfirst, RP)].set(
        True, mode="drop")
    isb = isb.at[jnp.where(dlast < RP, dlast, RP)].set(True, mode="drop")
    isb = isb & nonempty
    in1 = nonempty & ~isb
    slot1 = (startc // BS) * BS + rank[startc]
    ptr1 = jnp.where(in1, slot1, zslot[r % L2])
    r2of = jnp.zeros((RP,), i32).at[jnp.where(l2d < RP, l2d, RP)].set(
        ranks2, mode="drop")
    ptr2 = jnp.where(isb, r2of, ranks2[2 * nb + (r % pad2)])
    return (rank.reshape(Pp, 1), l2gptr, ranks2.reshape(L2, 1),
            ptr1, ptr2)


def _segsum(data, gptr, plan):
    ranks, l2gptr, ranks2, ptr1, ptr2 = plan
    xs = _sc_gather(data, gptr)
    lvl1 = _tc_onehot(xs, ranks, BS)
    l2in = _sc_gather(lvl1, l2gptr)
    lvl2 = _tc_onehot(l2in, ranks2, L2)
    s1 = _sc_gather(lvl1, ptr1)
    s2 = _sc_gather(lvl2, ptr2)
    return s1, s2


def _sort_pairs(g, d, Pp, zr0, nzr):
    i32 = jnp.int32
    order = jnp.argsort(d)
    gs = g.astype(i32)[order]
    ds = d.astype(i32)[order]
    pad = Pp - g.shape[0]
    zs = zr0 + (jnp.arange(pad, dtype=i32) % nzr)
    gptr = jnp.concatenate([gs, zs])
    dsort = jnp.concatenate([ds, jnp.full((pad,), SENT, i32)])
    return gptr, dsort


def kernel(x, edge_attr, edge_index0, edge_index1, n_e, e_order, batch,
           atom_emb, bond_emb, W1a, b1a, W1b, b1b, W2a, b2a, W2b, b2b,
           W3a, b3a, W3b, b3b, W4a, b4a, W4b, b4b, Wo1, bo1, Wo2, bo2):
    f32, i32 = jnp.float32, jnp.int32
    V, E = edge_index0.astype(i32), edge_index1.astype(i32)
    ZR, NZ = NV, RP - NV

    at_ext = jnp.concatenate([atom_emb.reshape(900, HD).astype(f32),
                              jnp.zeros((124, HD), f32)])
    x_p = jnp.pad(x.astype(i32), ((0, RP - NV), (0, 0)))
    g9 = x_p + (jnp.arange(9, dtype=i32) * 100)[None, :]
    zs_n = 900 + (jnp.arange(RP, dtype=i32) % 124)[:, None]
    g9 = jnp.where((jnp.arange(RP) < NV)[:, None], g9, zs_n).reshape(-1)

    bd_ext = jnp.concatenate([bond_emb.astype(f32), jnp.zeros((58, HD), f32)])
    gbd = jnp.concatenate([edge_attr[:, 0].astype(i32),
                           6 + (jnp.arange(RP - NV, dtype=i32) % 58)])

    gE, dE = _sort_pairs(V, E, 32768, ZR, NZ)
    planE = _plan(dE, 32768)
    gV, dV = _sort_pairs(E, V, 32768, ZR, NZ)
    planV = _plan(dV, 32768)
    dB = jnp.concatenate([batch.astype(i32),
                          jnp.full((RP - NV,), SENT, i32)])
    planB = _plan(dB, RP)

    eo_p = jnp.pad(e_order.astype(i32), (0, RP - NV)).reshape(RP, 1)
    b1a_, b1b_ = b1a.reshape(1, HD), b1b.reshape(1, HD)
    b2a_, b2b_ = b2a.reshape(1, HD), b2b.reshape(1, HD)
    b3a_, b3b_ = b3a.reshape(1, HD), b3b.reshape(1, HD)
    b4a_, b4b_ = b4a.reshape(1, HD), b4b.reshape(1, HD)
    bo1_, bo2_ = bo1.reshape(1, HD), bo2.reshape(1, 1)

    xh = _tc_sum9(_sc_gather(at_ext, g9))
    eh = _sc_gather(bd_ext, gbd)
    A = _tc_mlp(xh, None, None, None, W1a, b1a_, W1b, b1b_)
    for i in range(3):
        e1, e2 = _segsum(A, gE, planE)
        eh_new, Bm = _tc_mlp(eh, e1, e2, None, W2a, b2a_, W2b, b2b_,
                             second=(W3a, b3a_, W3b, b3b_),
                             relu_h=(i > 0), relu2_in=False)
        v1, v2 = _segsum(Bm, gV, planV)
        if i < 2:
            xh, A = _tc_mlp(xh, v1, v2, None, W4a, b4a_, W4b, b4b_,
                            second=(W1a, b1a_, W1b, b1b_),
                            relu_h=(i > 0), relu2_in=True)
        else:
            xh = _tc_mlp(xh, v1, v2, None, W4a, b4a_, W4b, b4b_, relu_h=True)
        eh = eh_new

    ranksB, l2gB, ranks2B, ptr1B, ptr2B = planB
    lvl1 = _tc_onehot(xh, ranksB, BS)
    lvl2 = _tc_onehot(_sc_gather(lvl1, l2gB), ranks2B, L2)
    g1 = _sc_gather(lvl1, ptr1B)
    g2 = _sc_gather(lvl2, ptr2B)
    out = _tc_mlp(g1, eh, None, eo_p, Wo1, bo1_, Wo2, bo2_, h2=g2)
    return out[:NV, 0]


# speedup vs baseline: 3.0295x; 1.0019x over previous
import functools

import jax
import jax.numpy as jnp
from jax import lax
from jax.experimental import pallas as pl
from jax.experimental.pallas import tpu as pltpu
from jax.experimental.pallas import tpu_sc as plsc

NV = 10000
RP = 10240
HD = 256
BN = 1024
BS = 128
L2 = 1024
NW = 32
SENT = 1 << 30


def _sc_gather(data, gidx):
    Pp = gidx.shape[0]
    per_tile = Pp // NW
    ch = per_tile if per_tile <= 128 else 128
    while per_tile % ch:
        ch //= 2
    nch = per_tile // ch
    mesh = plsc.VectorSubcoreMesh(core_axis_name="c", subcore_axis_name="s",
                                  num_cores=2, num_subcores=16)

    nbuf = 3 if ch == 128 else (4 if nch >= 4 else 2)
    mems = []
    for _ in range(nbuf):
        mems.append(pltpu.VMEM((ch, HD), jnp.float32))
    for _ in range(nbuf):
        mems.append(pltpu.VMEM((ch,), jnp.int32))
    for _ in range(nbuf):
        mems.append(pltpu.SemaphoreType.DMA)

    @functools.partial(
        pl.kernel,
        mesh=mesh,
        out_type=jax.ShapeDtypeStruct((Pp, HD), jnp.float32),
        scratch_types=mems,
    )
    def k(data_h, g_h, out_h, *bufs):
        c = lax.axis_index("c")
        t = lax.axis_index("s")
        w = c * 16 + t
        dbs = bufs[:nbuf]
        gbs = bufs[nbuf:2 * nbuf]
        sems = bufs[2 * nbuf:]
        base = w * per_tile
        descs = {}

        def start(j):
            b = j % nbuf
            pltpu.sync_copy(g_h.at[pl.ds(base + j * ch, ch)], gbs[b])
            descs[b] = pltpu.async_copy(data_h.at[gbs[b]], dbs[b], sems[b])

        for j in range(min(nbuf - 1, nch)):
            start(j)
        for j in range(nch):
            if j + nbuf - 1 < nch:
                start(j + nbuf - 1)
            b = j % nbuf
            descs[b].wait()
            pltpu.sync_copy(dbs[b], out_h.at[pl.ds(base + j * ch, ch)])

    return k(data, gidx)


def _sc_gather2(dataA, gidxA, dataB, gidxB):
    Pp = gidxA.shape[0]
    per_tile = Pp // NW
    ch = per_tile if per_tile <= 128 else 128
    while per_tile % ch:
        ch //= 2
    nch = per_tile // ch
    mesh = plsc.VectorSubcoreMesh(core_axis_name="c", subcore_axis_name="s",
                                  num_cores=2, num_subcores=16)

    @functools.partial(
        pl.kernel,
        mesh=mesh,
        out_type=[jax.ShapeDtypeStruct((Pp, HD), jnp.float32),
                  jax.ShapeDtypeStruct((Pp, HD), jnp.float32)],
        scratch_types=[
            pltpu.VMEM((ch, HD), jnp.float32),
            pltpu.VMEM((ch, HD), jnp.float32),
            pltpu.VMEM((ch,), jnp.int32),
            pltpu.VMEM((ch,), jnp.int32),
            pltpu.SemaphoreType.DMA,
            pltpu.SemaphoreType.DMA,
        ],
    )
    def k(dA_h, gA_h, dB_h, gB_h, oA_h, oB_h, dbA, dbB, gbA, gbB, semA, semB):
        c = lax.axis_index("c")
        t = lax.axis_index("s")
        w = c * 16 + t
        base = w * per_tile
        for j in range(nch):
            off = base + j * ch
            pltpu.sync_copy(gA_h.at[pl.ds(off, ch)], gbA)
            pltpu.sync_copy(gB_h.at[pl.ds(off, ch)], gbB)
            da = pltpu.async_copy(dA_h.at[gbA], dbA, semA)
            db_ = pltpu.async_copy(dB_h.at[gbB], dbB, semB)
            da.wait()
            pltpu.sync_copy(dbA, oA_h.at[pl.ds(off, ch)])
            db_.wait()
            pltpu.sync_copy(dbB, oB_h.at[pl.ds(off, ch)])

    return k(dataA, gidxA, dataB, gidxB)


def _tc_onehot(x, ranks, bs):
    Pp = x.shape[0]

    def body(x_ref, r_ref, o_ref):
        xv = x_ref[...]
        rv = r_ref[...]
        kcol = lax.broadcasted_iota(jnp.int32, (bs, bs), 1)
        oh = (rv == kcol).astype(jnp.float32)
        o_ref[...] = lax.dot_general(
            oh, xv, (((0,), (0,)), ((), ())),
            preferred_element_type=jnp.float32)

    return pl.pallas_call(
        body,
        grid=(Pp // bs,),
        in_specs=[pl.BlockSpec((bs, HD), lambda i: (i, 0)),
                  pl.BlockSpec((bs, 1), lambda i: (i, 0))],
        out_specs=pl.BlockSpec((bs, HD), lambda i: (i, 0)),
        out_shape=jax.ShapeDtypeStruct((Pp, HD), jnp.float32),
        compiler_params=pltpu.CompilerParams(
            dimension_semantics=("arbitrary",)),
    )(x, ranks)


def _tc_sum9(x9):
    def body(x_ref, o_ref):
        o_ref[...] = jnp.sum(x_ref[...].reshape(BN, 9, HD), axis=1)

    return pl.pallas_call(
        body,
        grid=(RP // BN,),
        in_specs=[pl.BlockSpec((9 * BN, HD), lambda i: (i, 0))],
        out_specs=pl.BlockSpec((BN, HD), lambda i: (i, 0)),
        out_shape=jax.ShapeDtypeStruct((RP, HD), jnp.float32),
        compiler_params=pltpu.CompilerParams(
            dimension_semantics=("arbitrary",)),
    )(x9)


def _tc_mlp(h, s, s2, eo, Wa, ba, Wb, bb, second=None,
            relu_h=False, relu2_in=False, h2=None):
    has_s = s is not None
    has_s2 = s2 is not None
    has_h2 = h2 is not None
    has_eo = eo is not None
    has2 = second is not None
    od = Wb.shape[1]

    def body(*refs):
        it = iter(refs)
        h_ref = next(it)
        h2_ref = next(it) if has_h2 else None
        s_ref = next(it) if has_s else None
        s2_ref = next(it) if has_s2 else None
        eo_ref = next(it) if has_eo else None
        Wa_ref, ba_ref, Wb_ref, bb_ref = next(it), next(it), next(it), next(it)
        if has2:
            W2a_ref, b2a_ref, W2b_ref, b2b_ref = (next(it), next(it),
                                                  next(it), next(it))
        o1_ref = next(it)
        o2_ref = next(it) if has2 else None

        i = pl.program_id(0)
        hv = h_ref[...]
        if has_h2:
            hv = hv + h2_ref[...]
        if relu_h:
            hv = jnp.maximum(hv, 0.0)
        if has_s:
            sv = s_ref[...]
            if has_s2:
                sv = sv + s2_ref[...]
            if has_eo:
                sv = sv * (eo_ref[...] > 2).astype(jnp.float32)
            hv = jnp.concatenate([hv, sv], axis=1)
        z = jnp.maximum(
            jnp.dot(hv, Wa_ref[...], preferred_element_type=jnp.float32)
            + ba_ref[...], 0.0)
        y1 = jnp.dot(z, Wb_ref[...],
                     preferred_element_type=jnp.float32) + bb_ref[...]
        rows = i * BN + lax.broadcasted_iota(jnp.int32, (BN, 1), 0)
        valid = rows < NV
        o1_ref[...] = jnp.where(valid, y1, 0.0)
        if has2:
            t = jnp.maximum(y1, 0.0) if relu2_in else y1
            z2 = jnp.maximum(
                jnp.dot(t, W2a_ref[...], preferred_element_type=jnp.float32)
                + b2a_ref[...], 0.0)
            y2 = jnp.dot(z2, W2b_ref[...],
                         preferred_element_type=jnp.float32) + b2b_ref[...]
            o2_ref[...] = jnp.where(valid, y2, 0.0)

    def row_spec(a):
        nd = a.ndim
        return pl.BlockSpec((BN,) + a.shape[1:],
                            lambda i, _nd=nd: (i,) + (0,) * (_nd - 1))

    def full_spec(a):
        nd = a.ndim
        return pl.BlockSpec(a.shape, lambda i, _nd=nd: (0,) * _nd)

    args = [h]
    specs = [row_spec(h)]
    for extra, flag in ((h2, has_h2), (s, has_s), (s2, has_s2), (eo, has_eo)):
        if flag:
            args.append(extra)
            specs.append(row_spec(extra))
    for w in (Wa, ba, Wb, bb):
        args.append(w)
        specs.append(full_spec(w))
    if has2:
        for w in second:
            args.append(w)
            specs.append(full_spec(w))

    out_shape = [jax.ShapeDtypeStruct((RP, od), jnp.float32)]
    out_specs = [pl.BlockSpec((BN, od), lambda i: (i, 0))]
    if has2:
        out_shape.append(jax.ShapeDtypeStruct((RP, HD), jnp.float32))
        out_specs.append(pl.BlockSpec((BN, HD), lambda i: (i, 0)))

    res = pl.pallas_call(
        body,
        grid=(RP // BN,),
        in_specs=specs,
        out_specs=out_specs,
        out_shape=out_shape,
        compiler_params=pltpu.CompilerParams(
            dimension_semantics=("arbitrary",)),
    )(*args)
    return res if has2 else res[0]


def _plan(dsort, Pp):
    i32 = jnp.int32
    nb = Pp // BS
    pidx = jnp.arange(Pp, dtype=i32)
    first = jnp.concatenate([jnp.ones((1,), bool), dsort[1:] != dsort[:-1]])
    newf = first | (pidx % BS == 0)
    cs = jnp.cumsum(newf.astype(i32))
    rank = cs - cs[(pidx // BS) * BS]

    bstart = jnp.arange(nb, dtype=i32) * BS
    bend = bstart + BS - 1
    dfirst = dsort[bstart]
    dlast = dsort[bend]
    rlast = rank[bend]
    single = dfirst == dlast

    zslot = (nb - 1) * BS + 1 + (jnp.arange(L2, dtype=i32) % (BS - 1))

    l2d = jnp.stack([dfirst, dlast], 1).reshape(-1)
    l2g = jnp.stack([jnp.where(single, zslot[:nb], bstart),
                     bstart + rlast], 1).reshape(-1)
    pad2 = L2 - 2 * nb
    l2d = jnp.concatenate([l2d, SENT + 1 + jnp.arange(pad2, dtype=i32)])
    l2gptr = jnp.concatenate([l2g, zslot[:pad2]])
    f2 = jnp.concatenate([jnp.ones((1,), bool), l2d[1:] != l2d[:-1]])
    cs2 = jnp.cumsum(f2.astype(i32))
    ranks2 = cs2 - 1

    r = jnp.arange(RP, dtype=i32)
    start = jnp.searchsorted(dsort, r).astype(i32)
    end = jnp.searchsorted(dsort, r, side="right").astype(i32)
    nonempty = end > start
    startc = jnp.minimum(start, Pp - 1)
    isb = jnp.zeros((RP,), bool).at[jnp.where(dfirst < RP, dfirst, RP)].set(
        True, mode="drop")
    isb = isb.at[jnp.where(dlast < RP, dlast, RP)].set(True, mode="drop")
    isb = isb & nonempty
    in1 = nonempty & ~isb
    slot1 = (startc // BS) * BS + rank[startc]
    ptr1 = jnp.where(in1, slot1, zslot[r % L2])
    r2of = jnp.zeros((RP,), i32).at[jnp.where(l2d < RP, l2d, RP)].set(
        ranks2, mode="drop")
    ptr2 = jnp.where(isb, r2of, ranks2[2 * nb + (r % pad2)])
    return (rank.reshape(Pp, 1), l2gptr, ranks2.reshape(L2, 1),
            ptr1, ptr2)


def _segsum(data, gptr, plan):
    ranks, l2gptr, ranks2, ptr1, ptr2 = plan
    xs = _sc_gather(data, gptr)
    lvl1 = _tc_onehot(xs, ranks, BS)
    l2in = _sc_gather(lvl1, l2gptr)
    lvl2 = _tc_onehot(l2in, ranks2, L2)
    return _sc_gather2(lvl1, ptr1, lvl2, ptr2)


def _sort_pairs(g, d, Pp, zr0, nzr):
    i32 = jnp.int32
    order = jnp.argsort(d)
    gs = g.astype(i32)[order]
    ds = d.astype(i32)[order]
    pad = Pp - g.shape[0]
    zs = zr0 + (jnp.arange(pad, dtype=i32) % nzr)
    gptr = jnp.concatenate([gs, zs])
    dsort = jnp.concatenate([ds, jnp.full((pad,), SENT, i32)])
    return gptr, dsort


def kernel(x, edge_attr, edge_index0, edge_index1, n_e, e_order, batch,
           atom_emb, bond_emb, W1a, b1a, W1b, b1b, W2a, b2a, W2b, b2b,
           W3a, b3a, W3b, b3b, W4a, b4a, W4b, b4b, Wo1, bo1, Wo2, bo2):
    f32, i32 = jnp.float32, jnp.int32
    V, E = edge_index0.astype(i32), edge_index1.astype(i32)
    ZR, NZ = NV, RP - NV

    at_ext = jnp.concatenate([atom_emb.reshape(900, HD).astype(f32),
                              jnp.zeros((124, HD), f32)])
    x_p = jnp.pad(x.astype(i32), ((0, RP - NV), (0, 0)))
    g9 = x_p + (jnp.arange(9, dtype=i32) * 100)[None, :]
    zs_n = 900 + (jnp.arange(RP, dtype=i32) % 124)[:, None]
    g9 = jnp.where((jnp.arange(RP) < NV)[:, None], g9, zs_n).reshape(-1)

    bd_ext = jnp.concatenate([bond_emb.astype(f32), jnp.zeros((58, HD), f32)])
    gbd = jnp.concatenate([edge_attr[:, 0].astype(i32),
                           6 + (jnp.arange(RP - NV, dtype=i32) % 58)])

    gE, dE = _sort_pairs(V, E, 32768, ZR, NZ)
    planE = _plan(dE, 32768)
    gV, dV = _sort_pairs(E, V, 32768, ZR, NZ)
    planV = _plan(dV, 32768)
    dB = jnp.concatenate([batch.astype(i32),
                          jnp.full((RP - NV,), SENT, i32)])
    planB = _plan(dB, RP)

    eo_p = jnp.pad(e_order.astype(i32), (0, RP - NV)).reshape(RP, 1)
    b1a_, b1b_ = b1a.reshape(1, HD), b1b.reshape(1, HD)
    b2a_, b2b_ = b2a.reshape(1, HD), b2b.reshape(1, HD)
    b3a_, b3b_ = b3a.reshape(1, HD), b3b.reshape(1, HD)
    b4a_, b4b_ = b4a.reshape(1, HD), b4b.reshape(1, HD)
    bo1_, bo2_ = bo1.reshape(1, HD), bo2.reshape(1, 1)

    xh = _tc_sum9(_sc_gather(at_ext, g9))
    eh = _sc_gather(bd_ext, gbd)
    A = _tc_mlp(xh, None, None, None, W1a, b1a_, W1b, b1b_)
    for i in range(3):
        e1, e2 = _segsum(A, gE, planE)
        eh_new, Bm = _tc_mlp(eh, e1, e2, None, W2a, b2a_, W2b, b2b_,
                             second=(W3a, b3a_, W3b, b3b_),
                             relu_h=(i > 0), relu2_in=False)
        v1, v2 = _segsum(Bm, gV, planV)
        if i < 2:
            xh, A = _tc_mlp(xh, v1, v2, None, W4a, b4a_, W4b, b4b_,
                            second=(W1a, b1a_, W1b, b1b_),
                            relu_h=(i > 0), relu2_in=True)
        else:
            xh = _tc_mlp(xh, v1, v2, None, W4a, b4a_, W4b, b4b_, relu_h=True)
        eh = eh_new

    ranksB, l2gB, ranks2B, ptr1B, ptr2B = planB
    lvl1 = _tc_onehot(xh, ranksB, BS)
    lvl2 = _tc_onehot(_sc_gather(lvl1, l2gB), ranks2B, L2)
    g1, g2 = _sc_gather2(lvl1, ptr1B, lvl2, ptr2B)
    out = _tc_mlp(g1, eh, None, eo_p, Wo1, bo1_, Wo2, bo2_, h2=g2)
    return out[:NV, 0]
